# Initial kernel scaffold; baseline (speedup 1.0000x reference)
import jax
import jax.numpy as jnp
from jax.experimental import pallas as pl


def kernel(x, edge_index, W1, b1, bn1_g, bn1_b, Wc1, bc1, Wc2, bc2, We1, be1, bne_g, bne_b, We2, be2):
    raise NotImplementedError("write your pallas kernel here")


# speedup vs baseline: 9.3696x; 9.3696x over previous
import functools

import jax
import jax.numpy as jnp
from jax import lax
from jax.experimental import pallas as pl
from jax.experimental.pallas import tpu as pltpu
from jax.experimental.pallas import tpu_sc as plsc

N = 10000
E = 320000
F_IN = 128
H = 64
W128 = 128
EPS = 1e-5

NC = 2
NS = 16
NW = NC * NS
EB = 128
NEBP = 2560
E_PAD = NEBP * EB
NBW = NEBP // NW
NHF = 2
NBW2 = NBW // NHF
PAD_ROWS = 16
NT = N + PAD_ROWS
NPS = 624
RB = 2000
F32 = jnp.float32


def _mesh():
    return plsc.VectorSubcoreMesh(
        core_axis_name="c", subcore_axis_name="s", num_cores=NC, num_subcores=NS
    )


def _writeback(acc_sp, cid, sid, p0_h, p1_h):
    base = pl.multiple_of(sid * NPS, 8)
    tail = NS * NPS

    @pl.when(cid == 0)
    def _():
        pltpu.sync_copy(acc_sp.at[pl.ds(base, NPS)], p0_h.at[pl.ds(base, NPS)])

        @pl.when(sid == NS - 1)
        def _():
            pltpu.sync_copy(acc_sp.at[pl.ds(tail, N - tail)],
                            p0_h.at[pl.ds(tail, N - tail)])

    @pl.when(cid == 1)
    def _():
        pltpu.sync_copy(acc_sp.at[pl.ds(base, NPS)], p1_h.at[pl.ds(base, NPS)])

        @pl.when(sid == NS - 1)
        def _():
            pltpu.sync_copy(acc_sp.at[pl.ds(tail, N - tail)],
                            p1_h.at[pl.ds(tail, N - tail)])


def _zero_acc(z_h, acc_sp, sid, sem):
    base = pl.multiple_of(sid * NPS, 8)
    cp = pltpu.async_copy(z_h.at[pl.ds(base, NPS)], acc_sp.at[pl.ds(base, NPS)], sem)
    tail = NS * NPS

    @pl.when(sid == NS - 1)
    def _():
        pltpu.sync_copy(z_h.at[pl.ds(tail, NT - tail)],
                        acc_sp.at[pl.ds(tail, NT - tail)])

    return cp


def _deg_sc(dst2, ones_blk, zeros128):
    @functools.partial(
        pl.kernel,
        mesh=_mesh(),
        out_type=(
            jax.ShapeDtypeStruct((N, W128), F32),
            jax.ShapeDtypeStruct((N, W128), F32),
        ),
        scratch_types=[
            pltpu.MemorySpace.VMEM_SHARED((NT, W128), F32),
            pltpu.VMEM((NBW2, EB), jnp.int32),
            pltpu.VMEM((EB, W128), F32),
            pltpu.SemaphoreType.DMA,
        ],
    )
    def body(dst_h, ones_h, z_h, d0_h, d1_h, deg_sp, didx, ones_v, sem):
        cid = lax.axis_index("c")
        sid = lax.axis_index("s")
        wid = sid * NC + cid
        start = pl.multiple_of(wid * NBW, 8)
        stage = _zero_acc(z_h, deg_sp, sid, sem)
        pltpu.sync_copy(ones_h, ones_v)
        stage.wait()
        plsc.subcore_barrier()

        for hf in range(NHF):
            hstart = pl.multiple_of(start + hf * NBW2, 8)
            pltpu.sync_copy(dst_h.at[pl.ds(hstart, NBW2)], didx)

            @pl.loop(0, NBW2)
            def _(it):
                pltpu.sync_copy(ones_v, deg_sp.at[didx.at[it]], add=True)

        plsc.subcore_barrier()
        _writeback(deg_sp, cid, sid, d0_h, d1_h)

    return body(dst2, ones_blk, zeros128)


def _conv_sc(y128, src2, dst2, zeros128):
    @functools.partial(
        pl.kernel,
        mesh=_mesh(),
        out_type=(
            jax.ShapeDtypeStruct((N, W128), F32),
            jax.ShapeDtypeStruct((N, W128), F32),
        ),
        scratch_types=[
            pltpu.MemorySpace.VMEM_SHARED((NT, W128), F32),
            pltpu.VMEM((NBW2, EB), jnp.int32),
            pltpu.VMEM((NBW2, EB), jnp.int32),
            pltpu.VMEM((EB, W128), F32),
            pltpu.SemaphoreType.DMA,
            pltpu.SemaphoreType.DMA,
        ],
    )
    def body(y_h, src_h, dst_h, z_h, p0_h, p1_h, acc_sp, sidx, didx,
             rows, sem_z, sem_g):
        cid = lax.axis_index("c")
        sid = lax.axis_index("s")
        wid = sid * NC + cid
        start = pl.multiple_of(wid * NBW, 8)
        stage = _zero_acc(z_h, acc_sp, sid, sem_z)
        stage.wait()
        plsc.subcore_barrier()

        for hf in range(NHF):
            hstart = pl.multiple_of(start + hf * NBW2, 8)
            pltpu.sync_copy(src_h.at[pl.ds(hstart, NBW2)], sidx)
            pltpu.sync_copy(dst_h.at[pl.ds(hstart, NBW2)], didx)

            @pl.loop(0, NBW2)
            def _(it):
                pltpu.async_copy(y_h.at[sidx.at[it]], rows, sem_g).wait()
                pltpu.sync_copy(rows, acc_sp.at[didx.at[it]], add=True)

        plsc.subcore_barrier()
        _writeback(acc_sp, cid, sid, p0_h, p1_h)

    return body(y128, src2, dst2, zeros128)


def _head_sc(ab, src2, dst2, w2):
    @functools.partial(
        pl.kernel,
        mesh=_mesh(),
        out_type=jax.ShapeDtypeStruct((E_PAD * 16,), F32),
        scratch_types=[
            pltpu.VMEM((NBW2, EB), jnp.int32),
            pltpu.VMEM((NBW2, EB), jnp.int32),
            pltpu.VMEM((EB, W128), F32),
            pltpu.VMEM((EB, W128), F32),
            pltpu.VMEM((EB * 16,), F32),
            pltpu.VMEM((H,), F32),
            pltpu.SemaphoreType.DMA,
            pltpu.SemaphoreType.DMA,
        ],
    )
    def body(ab_h, src_h, dst_h, w_h, out_h, ridx, cidx, arows, brows,
             souts, wv, sem_a, sem_b):
        cid = lax.axis_index("c")
        sid = lax.axis_index("s")
        wid = sid * NC + cid
        start = pl.multiple_of(wid * NBW, 8)
        pltpu.sync_copy(w_h, wv)
        wvecs = [wv[pl.ds(16 * j, 16)] for j in range(H // 16)]

        for hf in range(NHF):
            hstart = pl.multiple_of(start + hf * NBW2, 8)
            pltpu.sync_copy(src_h.at[pl.ds(hstart, NBW2)], ridx)
            pltpu.sync_copy(dst_h.at[pl.ds(hstart, NBW2)], cidx)

            @pl.loop(0, NBW2)
            def _(it):
                ca = pltpu.async_copy(ab_h.at[ridx.at[it]], arows, sem_a)
                cb = pltpu.async_copy(ab_h.at[cidx.at[it]], brows, sem_b)
                ca.wait()
                cb.wait()

                @pl.loop(0, EB)
                def _(e):
                    s = jnp.zeros((16,), F32)
                    for j in range(H // 16):
                        z = (arows[e, pl.ds(16 * j, 16)]
                             + brows[e, pl.ds(H + 16 * j, 16)])
                        s = s + jnp.maximum(z, 0.0) * wvecs[j]
                    souts[pl.ds(pl.multiple_of(e * 16, 16), 16)] = s

                pltpu.sync_copy(
                    souts,
                    out_h.at[
                        pl.ds(pl.multiple_of((hstart + it) * EB * 16, 128), EB * 16)
                    ],
                )

    return body(ab, src2, dst2, w2)


def _pad128(v):
    return jnp.concatenate([v, jnp.zeros_like(v)], axis=1)


def _tc1(x, w1f, b1f, wc1, d0, d1):
    def body(x_r, w1_r, b1_r, wc1_r, d0_r, d1_r, y1_r, dinv_r):
        h = jnp.maximum(
            jnp.dot(x_r[...], w1_r[...], preferred_element_type=F32) + b1_r[...],
            0.0,
        )
        deg = d0_r[...][:, 0:1] + d1_r[...][:, 0:1] + 1.0
        dinv = lax.rsqrt(deg)
        dinv_r[...] = dinv
        y1_r[...] = _pad128(
            jnp.dot(h, wc1_r[...], preferred_element_type=F32) * dinv
        )

    grid = (N // RB,)
    return pl.pallas_call(
        body,
        grid=grid,
        in_specs=[
            pl.BlockSpec((RB, F_IN), lambda i: (i, 0)),
            pl.BlockSpec((F_IN, H), lambda i: (0, 0)),
            pl.BlockSpec((H,), lambda i: (0,)),
            pl.BlockSpec((H, H), lambda i: (0, 0)),
            pl.BlockSpec((RB, W128), lambda i: (i, 0)),
            pl.BlockSpec((RB, W128), lambda i: (i, 0)),
        ],
        out_specs=[
            pl.BlockSpec((RB, W128), lambda i: (i, 0)),
            pl.BlockSpec((RB, 1), lambda i: (i, 0)),
        ],
        out_shape=[
            jax.ShapeDtypeStruct((N, W128), F32),
            jax.ShapeDtypeStruct((N, 1), F32),
        ],
    )(x, w1f, b1f, wc1, d0, d1)


def _tc2(p0, p1, y, dinv, bc, wnext):
    def body(p0_r, p1_r, y_r, dinv_r, bc_r, w_r, ynext_r):
        dinv = dinv_r[...]
        agg = (p0_r[...] + p1_r[...] + y_r[...])[:, :H]
        h = jnp.maximum(dinv * agg + bc_r[...], 0.0)
        ynext_r[...] = _pad128(
            jnp.dot(h, w_r[...], preferred_element_type=F32) * dinv
        )

    grid = (N // RB,)
    return pl.pallas_call(
        body,
        grid=grid,
        in_specs=[
            pl.BlockSpec((RB, W128), lambda i: (i, 0)),
            pl.BlockSpec((RB, W128), lambda i: (i, 0)),
            pl.BlockSpec((RB, W128), lambda i: (i, 0)),
            pl.BlockSpec((RB, 1), lambda i: (i, 0)),
            pl.BlockSpec((H,), lambda i: (0,)),
            pl.BlockSpec((H, H), lambda i: (0, 0)),
        ],
        out_specs=pl.BlockSpec((RB, W128), lambda i: (i, 0)),
        out_shape=jax.ShapeDtypeStruct((N, W128), F32),
    )(p0, p1, y, dinv, bc, wnext)


def _tc3(q0, q1, y2, dinv, bc2, wa, wb, cvec):
    def body(q0_r, q1_r, y_r, dinv_r, bc_r, wa_r, wb_r, c_r, h3_r, ab_r):
        dinv = dinv_r[...]
        agg = (q0_r[...] + q1_r[...] + y_r[...])[:, :H]
        h3 = jnp.maximum(dinv * agg + bc_r[...], 0.0)
        h3_r[...] = h3
        a2 = jnp.dot(h3, wa_r[...], preferred_element_type=F32) + c_r[...]
        b2 = jnp.dot(h3, wb_r[...], preferred_element_type=F32)
        ab_r[...] = jnp.concatenate([a2, b2], axis=1)

    grid = (N // RB,)
    return pl.pallas_call(
        body,
        grid=grid,
        in_specs=[
            pl.BlockSpec((RB, W128), lambda i: (i, 0)),
            pl.BlockSpec((RB, W128), lambda i: (i, 0)),
            pl.BlockSpec((RB, W128), lambda i: (i, 0)),
            pl.BlockSpec((RB, 1), lambda i: (i, 0)),
            pl.BlockSpec((H,), lambda i: (0,)),
            pl.BlockSpec((H, H), lambda i: (0, 0)),
            pl.BlockSpec((H, H), lambda i: (0, 0)),
            pl.BlockSpec((H,), lambda i: (0,)),
        ],
        out_specs=[
            pl.BlockSpec((RB, H), lambda i: (i, 0)),
            pl.BlockSpec((RB, W128), lambda i: (i, 0)),
        ],
        out_shape=[
            jax.ShapeDtypeStruct((N, H), F32),
            jax.ShapeDtypeStruct((N, W128), F32),
        ],
    )(q0, q1, y2, dinv, bc2, wa, wb, cvec)


def _tc4(s2, be2):
    EBR = 8192

    def body(s_r, b_r, out_r):
        t = jnp.sum(s_r[...], axis=1) + b_r[0]
        out_r[...] = (1.0 / (1.0 + jnp.exp(-t)))[:, None]

    grid = (E_PAD // EBR,)
    return pl.pallas_call(
        body,
        grid=grid,
        in_specs=[
            pl.BlockSpec((EBR, 16), lambda i: (i, 0)),
            pl.BlockSpec((1,), lambda i: (0,)),
        ],
        out_specs=pl.BlockSpec((EBR, 1), lambda i: (i, 0)),
        out_shape=jax.ShapeDtypeStruct((E_PAD, 1), F32),
    )(s2, be2)


def kernel(x, edge_index, W1, b1, bn1_g, bn1_b, Wc1, bc1, Wc2, bc2,
           We1, be1, bne_g, bne_b, We2, be2):
    npad = E_PAD - E
    spread = jnp.arange(npad, dtype=jnp.int32) % PAD_ROWS
    src2 = jnp.concatenate([edge_index[0], spread]).reshape(NEBP, EB)
    dst2 = jnp.concatenate([edge_index[1], N + spread]).reshape(NEBP, EB)

    s1 = bn1_g / jnp.sqrt(1.0 + EPS)
    w1f = W1 * s1[None, :]
    b1f = b1 * s1 + bn1_b
    sg = bne_g / jnp.sqrt(1.0 + EPS)
    wa = We1[:H] * sg[None, :]
    wb = We1[H:] * sg[None, :]
    cvec = be1 * sg + bne_b
    w2 = We2[:, 0]
    zeros128 = jnp.zeros((NT, W128), F32)
    ones_blk = jnp.ones((EB, W128), F32)

    d0, d1 = _deg_sc(dst2, ones_blk, zeros128)
    y1, dinv = _tc1(x, w1f, b1f, Wc1, d0, d1)
    p0, p1 = _conv_sc(y1, src2, dst2, zeros128)
    y2 = _tc2(p0, p1, y1, dinv, bc1, Wc2)
    q0, q1 = _conv_sc(y2, src2, dst2, zeros128)
    h3, ab = _tc3(q0, q1, y2, dinv, bc2, wa, wb, cvec)
    s_flat = _head_sc(ab, src2, dst2, w2)
    pred = _tc4(s_flat.reshape(E_PAD, 16), be2)
    return (h3, pred[:E])


# speedup vs baseline: 11.3060x; 1.2067x over previous
import functools

import jax
import jax.numpy as jnp
from jax import lax
from jax.experimental import pallas as pl
from jax.experimental.pallas import tpu as pltpu
from jax.experimental.pallas import tpu_sc as plsc

N = 10000
E = 320000
F_IN = 128
H = 64
W128 = 128
EPS = 1e-5

NC = 2
NS = 16
NW = NC * NS
EB = 128
NEBP = 2560
E_PAD = NEBP * EB
NBW = NEBP // NW
NHF = 2
NBW2 = NBW // NHF
PAD_ROWS = 16
NT = N + PAD_ROWS
NPS = 624
RB = 2000
F32 = jnp.float32


def _mesh():
    return plsc.VectorSubcoreMesh(
        core_axis_name="c", subcore_axis_name="s", num_cores=NC, num_subcores=NS
    )


def _writeback(acc_sp, cid, sid, p0_h, p1_h):
    base = pl.multiple_of(sid * NPS, 8)
    tail = NS * NPS

    @pl.when(cid == 0)
    def _():
        pltpu.sync_copy(acc_sp.at[pl.ds(base, NPS)], p0_h.at[pl.ds(base, NPS)])

        @pl.when(sid == NS - 1)
        def _():
            pltpu.sync_copy(acc_sp.at[pl.ds(tail, N - tail)],
                            p0_h.at[pl.ds(tail, N - tail)])

    @pl.when(cid == 1)
    def _():
        pltpu.sync_copy(acc_sp.at[pl.ds(base, NPS)], p1_h.at[pl.ds(base, NPS)])

        @pl.when(sid == NS - 1)
        def _():
            pltpu.sync_copy(acc_sp.at[pl.ds(tail, N - tail)],
                            p1_h.at[pl.ds(tail, N - tail)])


def _zero_acc(z_h, acc_sp, sid, sem):
    base = pl.multiple_of(sid * NPS, 8)
    cp = pltpu.async_copy(z_h.at[pl.ds(base, NPS)], acc_sp.at[pl.ds(base, NPS)], sem)
    tail = NS * NPS

    @pl.when(sid == NS - 1)
    def _():
        pltpu.sync_copy(z_h.at[pl.ds(tail, NT - tail)],
                        acc_sp.at[pl.ds(tail, NT - tail)])

    return cp


def _deg_sc(dst2, ones_blk, zeros128):
    @functools.partial(
        pl.kernel,
        mesh=_mesh(),
        out_type=(
            jax.ShapeDtypeStruct((N, W128), F32),
            jax.ShapeDtypeStruct((N, W128), F32),
        ),
        scratch_types=[
            pltpu.MemorySpace.VMEM_SHARED((NT, W128), F32),
            pltpu.VMEM((NBW2, EB), jnp.int32),
            pltpu.VMEM((EB, W128), F32),
            pltpu.SemaphoreType.DMA,
        ],
    )
    def body(dst_h, ones_h, z_h, d0_h, d1_h, deg_sp, didx, ones_v, sem):
        cid = lax.axis_index("c")
        sid = lax.axis_index("s")
        wid = sid * NC + cid
        start = pl.multiple_of(wid * NBW, 8)
        stage = _zero_acc(z_h, deg_sp, sid, sem)
        pltpu.sync_copy(ones_h, ones_v)
        stage.wait()
        plsc.subcore_barrier()

        for hf in range(NHF):
            hstart = pl.multiple_of(start + hf * NBW2, 8)
            pltpu.sync_copy(dst_h.at[pl.ds(hstart, NBW2)], didx)

            @pl.loop(0, NBW2)
            def _(it):
                pltpu.async_copy(ones_v, deg_sp.at[didx.at[it]], sem, add=True)

            @pl.loop(0, NBW2)
            def _(it):
                pltpu.make_async_copy(ones_v, deg_sp.at[didx.at[it]], sem).wait()

        plsc.subcore_barrier()
        _writeback(deg_sp, cid, sid, d0_h, d1_h)

    return body(dst2, ones_blk, zeros128)


def _conv_sc(y128, src2, dst2, zeros128):
    @functools.partial(
        pl.kernel,
        mesh=_mesh(),
        out_type=(
            jax.ShapeDtypeStruct((N, W128), F32),
            jax.ShapeDtypeStruct((N, W128), F32),
        ),
        scratch_types=[
            pltpu.MemorySpace.VMEM_SHARED((NT, W128), F32),
            pltpu.VMEM((NBW2, EB), jnp.int32),
            pltpu.VMEM((NBW2, EB), jnp.int32),
            pltpu.VMEM((EB, W128), F32),
            pltpu.VMEM((EB, W128), F32),
            pltpu.SemaphoreType.DMA,
            pltpu.SemaphoreType.DMA,
            pltpu.SemaphoreType.DMA,
        ],
    )
    def body(y_h, src_h, dst_h, z_h, p0_h, p1_h, acc_sp, sidx, didx,
             rows0, rows1, sem_z, sem_g, sem_h):
        cid = lax.axis_index("c")
        sid = lax.axis_index("s")
        wid = sid * NC + cid
        start = pl.multiple_of(wid * NBW, 8)
        stage = _zero_acc(z_h, acc_sp, sid, sem_z)
        stage.wait()
        plsc.subcore_barrier()

        for hf in range(NHF):
            hstart = pl.multiple_of(start + hf * NBW2, 8)
            pltpu.sync_copy(src_h.at[pl.ds(hstart, NBW2)], sidx)
            pltpu.sync_copy(dst_h.at[pl.ds(hstart, NBW2)], didx)
            pltpu.async_copy(y_h.at[sidx.at[0]], rows0, sem_g)

            @pl.loop(0, NBW2, step=2)
            def _(it):
                pltpu.make_async_copy(y_h.at[sidx.at[it]], rows0, sem_g).wait()
                pltpu.async_copy(y_h.at[sidx.at[it + 1]], rows1, sem_h)
                pltpu.sync_copy(rows0, acc_sp.at[didx.at[it]], add=True)
                pltpu.make_async_copy(y_h.at[sidx.at[it + 1]], rows1, sem_h).wait()

                @pl.when(it < NBW2 - 2)
                def _():
                    pltpu.async_copy(y_h.at[sidx.at[it + 2]], rows0, sem_g)

                pltpu.sync_copy(rows1, acc_sp.at[didx.at[it + 1]], add=True)

        plsc.subcore_barrier()
        _writeback(acc_sp, cid, sid, p0_h, p1_h)

    return body(y128, src2, dst2, zeros128)


def _head_sc(ab, src2, dst2, w2):
    @functools.partial(
        pl.kernel,
        mesh=_mesh(),
        out_type=jax.ShapeDtypeStruct((E_PAD * 16,), F32),
        scratch_types=[
            pltpu.VMEM((NBW2, EB), jnp.int32),
            pltpu.VMEM((NBW2, EB), jnp.int32),
            pltpu.VMEM((EB, W128), F32),
            pltpu.VMEM((EB, W128), F32),
            pltpu.VMEM((EB, W128), F32),
            pltpu.VMEM((EB, W128), F32),
            pltpu.VMEM((EB * 16,), F32),
            pltpu.VMEM((H,), F32),
            pltpu.SemaphoreType.DMA,
            pltpu.SemaphoreType.DMA,
            pltpu.SemaphoreType.DMA,
            pltpu.SemaphoreType.DMA,
        ],
    )
    def body(ab_h, src_h, dst_h, w_h, out_h, ridx, cidx, a0, b0, a1, b1,
             souts, wv, sem_a0, sem_b0, sem_a1, sem_b1):
        cid = lax.axis_index("c")
        sid = lax.axis_index("s")
        wid = sid * NC + cid
        start = pl.multiple_of(wid * NBW, 8)
        pltpu.sync_copy(w_h, wv)
        wvecs = [wv[pl.ds(16 * j, 16)] for j in range(H // 16)]

        def compute_block(arows, brows, blk):
            @pl.loop(0, EB)
            def _(e):
                s = jnp.zeros((16,), F32)
                for j in range(H // 16):
                    z = (arows[e, pl.ds(16 * j, 16)]
                         + brows[e, pl.ds(H + 16 * j, 16)])
                    s = s + jnp.maximum(z, 0.0) * wvecs[j]
                souts[pl.ds(pl.multiple_of(e * 16, 16), 16)] = s

            pltpu.sync_copy(
                souts,
                out_h.at[pl.ds(pl.multiple_of(blk * EB * 16, 128), EB * 16)],
            )

        for hf in range(NHF):
            hstart = pl.multiple_of(start + hf * NBW2, 8)
            pltpu.sync_copy(src_h.at[pl.ds(hstart, NBW2)], ridx)
            pltpu.sync_copy(dst_h.at[pl.ds(hstart, NBW2)], cidx)
            pltpu.async_copy(ab_h.at[ridx.at[0]], a0, sem_a0)
            pltpu.async_copy(ab_h.at[cidx.at[0]], b0, sem_b0)

            @pl.loop(0, NBW2, step=2)
            def _(it):
                pltpu.make_async_copy(ab_h.at[ridx.at[it]], a0, sem_a0).wait()
                pltpu.make_async_copy(ab_h.at[cidx.at[it]], b0, sem_b0).wait()
                pltpu.async_copy(ab_h.at[ridx.at[it + 1]], a1, sem_a1)
                pltpu.async_copy(ab_h.at[cidx.at[it + 1]], b1, sem_b1)
                compute_block(a0, b0, hstart + it)
                pltpu.make_async_copy(ab_h.at[ridx.at[it + 1]], a1, sem_a1).wait()
                pltpu.make_async_copy(ab_h.at[cidx.at[it + 1]], b1, sem_b1).wait()

                @pl.when(it < NBW2 - 2)
                def _():
                    pltpu.async_copy(ab_h.at[ridx.at[it + 2]], a0, sem_a0)
                    pltpu.async_copy(ab_h.at[cidx.at[it + 2]], b0, sem_b0)

                compute_block(a1, b1, hstart + it + 1)

    return body(ab, src2, dst2, w2)


def _pad128(v):
    return jnp.concatenate([v, jnp.zeros_like(v)], axis=1)


def _tc1(x, w1f, b1f, wc1, d0, d1):
    def body(x_r, w1_r, b1_r, wc1_r, d0_r, d1_r, y1_r, dinv_r):
        h = jnp.maximum(
            jnp.dot(x_r[...], w1_r[...], preferred_element_type=F32) + b1_r[...],
            0.0,
        )
        deg = d0_r[...][:, 0:1] + d1_r[...][:, 0:1] + 1.0
        dinv = lax.rsqrt(deg)
        dinv_r[...] = dinv
        y1_r[...] = _pad128(
            jnp.dot(h, wc1_r[...], preferred_element_type=F32) * dinv
        )

    grid = (N // RB,)
    return pl.pallas_call(
        body,
        grid=grid,
        in_specs=[
            pl.BlockSpec((RB, F_IN), lambda i: (i, 0)),
            pl.BlockSpec((F_IN, H), lambda i: (0, 0)),
            pl.BlockSpec((H,), lambda i: (0,)),
            pl.BlockSpec((H, H), lambda i: (0, 0)),
            pl.BlockSpec((RB, W128), lambda i: (i, 0)),
            pl.BlockSpec((RB, W128), lambda i: (i, 0)),
        ],
        out_specs=[
            pl.BlockSpec((RB, W128), lambda i: (i, 0)),
            pl.BlockSpec((RB, 1), lambda i: (i, 0)),
        ],
        out_shape=[
            jax.ShapeDtypeStruct((N, W128), F32),
            jax.ShapeDtypeStruct((N, 1), F32),
        ],
    )(x, w1f, b1f, wc1, d0, d1)


def _tc2(p0, p1, y, dinv, bc, wnext):
    def body(p0_r, p1_r, y_r, dinv_r, bc_r, w_r, ynext_r):
        dinv = dinv_r[...]
        agg = (p0_r[...] + p1_r[...] + y_r[...])[:, :H]
        h = jnp.maximum(dinv * agg + bc_r[...], 0.0)
        ynext_r[...] = _pad128(
            jnp.dot(h, w_r[...], preferred_element_type=F32) * dinv
        )

    grid = (N // RB,)
    return pl.pallas_call(
        body,
        grid=grid,
        in_specs=[
            pl.BlockSpec((RB, W128), lambda i: (i, 0)),
            pl.BlockSpec((RB, W128), lambda i: (i, 0)),
            pl.BlockSpec((RB, W128), lambda i: (i, 0)),
            pl.BlockSpec((RB, 1), lambda i: (i, 0)),
            pl.BlockSpec((H,), lambda i: (0,)),
            pl.BlockSpec((H, H), lambda i: (0, 0)),
        ],
        out_specs=pl.BlockSpec((RB, W128), lambda i: (i, 0)),
        out_shape=jax.ShapeDtypeStruct((N, W128), F32),
    )(p0, p1, y, dinv, bc, wnext)


def _tc3(q0, q1, y2, dinv, bc2, wa, wb, cvec):
    def body(q0_r, q1_r, y_r, dinv_r, bc_r, wa_r, wb_r, c_r, h3_r, ab_r):
        dinv = dinv_r[...]
        agg = (q0_r[...] + q1_r[...] + y_r[...])[:, :H]
        h3 = jnp.maximum(dinv * agg + bc_r[...], 0.0)
        h3_r[...] = h3
        a2 = jnp.dot(h3, wa_r[...], preferred_element_type=F32) + c_r[...]
        b2 = jnp.dot(h3, wb_r[...], preferred_element_type=F32)
        ab_r[...] = jnp.concatenate([a2, b2], axis=1)

    grid = (N // RB,)
    return pl.pallas_call(
        body,
        grid=grid,
        in_specs=[
            pl.BlockSpec((RB, W128), lambda i: (i, 0)),
            pl.BlockSpec((RB, W128), lambda i: (i, 0)),
            pl.BlockSpec((RB, W128), lambda i: (i, 0)),
            pl.BlockSpec((RB, 1), lambda i: (i, 0)),
            pl.BlockSpec((H,), lambda i: (0,)),
            pl.BlockSpec((H, H), lambda i: (0, 0)),
            pl.BlockSpec((H, H), lambda i: (0, 0)),
            pl.BlockSpec((H,), lambda i: (0,)),
        ],
        out_specs=[
            pl.BlockSpec((RB, H), lambda i: (i, 0)),
            pl.BlockSpec((RB, W128), lambda i: (i, 0)),
        ],
        out_shape=[
            jax.ShapeDtypeStruct((N, H), F32),
            jax.ShapeDtypeStruct((N, W128), F32),
        ],
    )(q0, q1, y2, dinv, bc2, wa, wb, cvec)


def _tc4(s2, be2):
    EBR = 8192

    def body(s_r, b_r, out_r):
        t = jnp.sum(s_r[...], axis=1) + b_r[0]
        out_r[...] = (1.0 / (1.0 + jnp.exp(-t)))[:, None]

    grid = (E_PAD // EBR,)
    return pl.pallas_call(
        body,
        grid=grid,
        in_specs=[
            pl.BlockSpec((EBR, 16), lambda i: (i, 0)),
            pl.BlockSpec((1,), lambda i: (0,)),
        ],
        out_specs=pl.BlockSpec((EBR, 1), lambda i: (i, 0)),
        out_shape=jax.ShapeDtypeStruct((E_PAD, 1), F32),
    )(s2, be2)


def kernel(x, edge_index, W1, b1, bn1_g, bn1_b, Wc1, bc1, Wc2, bc2,
           We1, be1, bne_g, bne_b, We2, be2):
    npad = E_PAD - E
    spread = jnp.arange(npad, dtype=jnp.int32) % PAD_ROWS
    src2 = jnp.concatenate([edge_index[0], spread]).reshape(NEBP, EB)
    dst2 = jnp.concatenate([edge_index[1], N + spread]).reshape(NEBP, EB)

    s1 = bn1_g / jnp.sqrt(1.0 + EPS)
    w1f = W1 * s1[None, :]
    b1f = b1 * s1 + bn1_b
    sg = bne_g / jnp.sqrt(1.0 + EPS)
    wa = We1[:H] * sg[None, :]
    wb = We1[H:] * sg[None, :]
    cvec = be1 * sg + bne_b
    w2 = We2[:, 0]
    zeros128 = jnp.zeros((NT, W128), F32)
    ones_blk = jnp.ones((EB, W128), F32)

    d0, d1 = _deg_sc(dst2, ones_blk, zeros128)
    y1, dinv = _tc1(x, w1f, b1f, Wc1, d0, d1)
    p0, p1 = _conv_sc(y1, src2, dst2, zeros128)
    y2 = _tc2(p0, p1, y1, dinv, bc1, Wc2)
    q0, q1 = _conv_sc(y2, src2, dst2, zeros128)
    h3, ab = _tc3(q0, q1, y2, dinv, bc2, wa, wb, cvec)
    s_flat = _head_sc(ab, src2, dst2, w2)
    pred = _tc4(s_flat.reshape(E_PAD, 16), be2)
    return (h3, pred[:E])


# speedup vs baseline: 11.6190x; 1.0277x over previous
import functools

import jax
import jax.numpy as jnp
from jax import lax
from jax.experimental import pallas as pl
from jax.experimental.pallas import tpu as pltpu
from jax.experimental.pallas import tpu_sc as plsc

N = 10000
E = 320000
F_IN = 128
H = 64
W128 = 128
EPS = 1e-5

NC = 2
NS = 16
NW = NC * NS
EB = 128
NEBP = 2560
E_PAD = NEBP * EB
NBW = NEBP // NW
NHF = 2
NBW2 = NBW // NHF
PAD_ROWS = 16
NT = N + PAD_ROWS
NPS = 624
RB = 2000
F32 = jnp.float32


def _mesh():
    return plsc.VectorSubcoreMesh(
        core_axis_name="c", subcore_axis_name="s", num_cores=NC, num_subcores=NS
    )


def _writeback(acc_sp, cid, sid, p0_h, p1_h):
    base = pl.multiple_of(sid * NPS, 8)
    tail = NS * NPS

    @pl.when(cid == 0)
    def _():
        pltpu.sync_copy(acc_sp.at[pl.ds(base, NPS)], p0_h.at[pl.ds(base, NPS)])

        @pl.when(sid == NS - 1)
        def _():
            pltpu.sync_copy(acc_sp.at[pl.ds(tail, N - tail)],
                            p0_h.at[pl.ds(tail, N - tail)])

    @pl.when(cid == 1)
    def _():
        pltpu.sync_copy(acc_sp.at[pl.ds(base, NPS)], p1_h.at[pl.ds(base, NPS)])

        @pl.when(sid == NS - 1)
        def _():
            pltpu.sync_copy(acc_sp.at[pl.ds(tail, N - tail)],
                            p1_h.at[pl.ds(tail, N - tail)])


def _zero_acc(z_h, acc_sp, sid, sem):
    base = pl.multiple_of(sid * NPS, 8)
    cp = pltpu.async_copy(z_h.at[pl.ds(base, NPS)], acc_sp.at[pl.ds(base, NPS)], sem)
    tail = NS * NPS

    @pl.when(sid == NS - 1)
    def _():
        pltpu.sync_copy(z_h.at[pl.ds(tail, NT - tail)],
                        acc_sp.at[pl.ds(tail, NT - tail)])

    return cp


def _deg_sc(dst2, ones_blk, zeros128):
    @functools.partial(
        pl.kernel,
        mesh=_mesh(),
        out_type=(
            jax.ShapeDtypeStruct((N, W128), F32),
            jax.ShapeDtypeStruct((N, W128), F32),
        ),
        scratch_types=[
            pltpu.MemorySpace.VMEM_SHARED((NT, W128), F32),
            pltpu.VMEM((NBW2, EB), jnp.int32),
            pltpu.VMEM((EB, W128), F32),
            pltpu.SemaphoreType.DMA,
        ],
    )
    def body(dst_h, ones_h, z_h, d0_h, d1_h, deg_sp, didx, ones_v, sem):
        cid = lax.axis_index("c")
        sid = lax.axis_index("s")
        wid = sid * NC + cid
        start = pl.multiple_of(wid * NBW, 8)
        stage = _zero_acc(z_h, deg_sp, sid, sem)
        pltpu.sync_copy(ones_h, ones_v)
        stage.wait()
        plsc.subcore_barrier()

        for hf in range(NHF):
            hstart = pl.multiple_of(start + hf * NBW2, 8)
            pltpu.sync_copy(dst_h.at[pl.ds(hstart, NBW2)], didx)

            @pl.loop(0, NBW2)
            def _(it):
                pltpu.async_copy(ones_v, deg_sp.at[didx.at[it]], sem, add=True)

            @pl.loop(0, NBW2)
            def _(it):
                pltpu.make_async_copy(ones_v, deg_sp.at[didx.at[it]], sem).wait()

        plsc.subcore_barrier()
        _writeback(deg_sp, cid, sid, d0_h, d1_h)

    return body(dst2, ones_blk, zeros128)


def _conv_sc(y128, src2, dst2, zeros128):
    @functools.partial(
        pl.kernel,
        mesh=_mesh(),
        out_type=(
            jax.ShapeDtypeStruct((N, W128), F32),
            jax.ShapeDtypeStruct((N, W128), F32),
        ),
        scratch_types=[
            pltpu.MemorySpace.VMEM_SHARED((NT, W128), F32),
            pltpu.VMEM((NBW2, EB), jnp.int32),
            pltpu.VMEM((NBW2, EB), jnp.int32),
            pltpu.VMEM((EB, W128), F32),
            pltpu.VMEM((EB, W128), F32),
            pltpu.SemaphoreType.DMA,
            pltpu.SemaphoreType.DMA,
            pltpu.SemaphoreType.DMA,
        ],
    )
    def body(y_h, src_h, dst_h, z_h, p0_h, p1_h, acc_sp, sidx, didx,
             rows0, rows1, sem_z, sem_g, sem_h):
        cid = lax.axis_index("c")
        sid = lax.axis_index("s")
        wid = sid * NC + cid
        start = pl.multiple_of(wid * NBW, 8)
        stage = _zero_acc(z_h, acc_sp, sid, sem_z)
        stage.wait()
        plsc.subcore_barrier()

        for hf in range(NHF):
            hstart = pl.multiple_of(start + hf * NBW2, 8)
            pltpu.sync_copy(src_h.at[pl.ds(hstart, NBW2)], sidx)
            pltpu.sync_copy(dst_h.at[pl.ds(hstart, NBW2)], didx)
            pltpu.async_copy(y_h.at[sidx.at[0]], rows0, sem_g)

            @pl.loop(0, NBW2, step=2)
            def _(it):
                pltpu.make_async_copy(y_h.at[sidx.at[it]], rows0, sem_g).wait()
                pltpu.async_copy(y_h.at[sidx.at[it + 1]], rows1, sem_h)
                pltpu.sync_copy(rows0, acc_sp.at[didx.at[it]], add=True)
                pltpu.make_async_copy(y_h.at[sidx.at[it + 1]], rows1, sem_h).wait()

                @pl.when(it < NBW2 - 2)
                def _():
                    pltpu.async_copy(y_h.at[sidx.at[it + 2]], rows0, sem_g)

                pltpu.sync_copy(rows1, acc_sp.at[didx.at[it + 1]], add=True)

        plsc.subcore_barrier()
        _writeback(acc_sp, cid, sid, p0_h, p1_h)

    return body(y128, src2, dst2, zeros128)


def _head_sc(ab, src2, dst2, w2):
    @functools.partial(
        pl.kernel,
        mesh=_mesh(),
        out_type=jax.ShapeDtypeStruct((E_PAD * 16,), F32),
        scratch_types=[
            pltpu.VMEM((NBW2, EB), jnp.int32),
            pltpu.VMEM((NBW2, EB), jnp.int32),
            pltpu.VMEM((EB, W128), F32),
            pltpu.VMEM((EB, W128), F32),
            pltpu.VMEM((EB, W128), F32),
            pltpu.VMEM((EB, W128), F32),
            pltpu.VMEM((EB * 16,), F32),
            pltpu.VMEM((H,), F32),
            pltpu.SemaphoreType.DMA,
            pltpu.SemaphoreType.DMA,
            pltpu.SemaphoreType.DMA,
            pltpu.SemaphoreType.DMA,
        ],
    )
    def body(ab_h, src_h, dst_h, w_h, out_h, ridx, cidx, a0, b0, a1, b1,
             souts, wv, sem_a0, sem_b0, sem_a1, sem_b1):
        cid = lax.axis_index("c")
        sid = lax.axis_index("s")
        wid = sid * NC + cid
        start = pl.multiple_of(wid * NBW, 8)
        pltpu.sync_copy(w_h, wv)
        wvecs = [wv[pl.ds(16 * j, 16)] for j in range(H // 16)]

        def compute_block(arows, brows, blk):
            @pl.loop(0, EB)
            def _(e):
                s = jnp.zeros((16,), F32)
                for j in range(H // 16):
                    z = (arows[e, pl.ds(16 * j, 16)]
                         + brows[e, pl.ds(H + 16 * j, 16)])
                    s = s + jnp.maximum(z, 0.0) * wvecs[j]
                souts[pl.ds(pl.multiple_of(e * 16, 16), 16)] = s

            pltpu.sync_copy(
                souts,
                out_h.at[pl.ds(pl.multiple_of(blk * EB * 16, 128), EB * 16)],
            )

        for hf in range(NHF):
            hstart = pl.multiple_of(start + hf * NBW2, 8)
            pltpu.sync_copy(src_h.at[pl.ds(hstart, NBW2)], ridx)
            pltpu.sync_copy(dst_h.at[pl.ds(hstart, NBW2)], cidx)
            pltpu.async_copy(ab_h.at[ridx.at[0]], a0, sem_a0)
            pltpu.async_copy(ab_h.at[cidx.at[0]], b0, sem_b0)

            @pl.loop(0, NBW2, step=2)
            def _(it):
                pltpu.make_async_copy(ab_h.at[ridx.at[it]], a0, sem_a0).wait()
                pltpu.make_async_copy(ab_h.at[cidx.at[it]], b0, sem_b0).wait()
                pltpu.async_copy(ab_h.at[ridx.at[it + 1]], a1, sem_a1)
                pltpu.async_copy(ab_h.at[cidx.at[it + 1]], b1, sem_b1)
                compute_block(a0, b0, hstart + it)
                pltpu.make_async_copy(ab_h.at[ridx.at[it + 1]], a1, sem_a1).wait()
                pltpu.make_async_copy(ab_h.at[cidx.at[it + 1]], b1, sem_b1).wait()

                @pl.when(it < NBW2 - 2)
                def _():
                    pltpu.async_copy(ab_h.at[ridx.at[it + 2]], a0, sem_a0)
                    pltpu.async_copy(ab_h.at[cidx.at[it + 2]], b0, sem_b0)

                compute_block(a1, b1, hstart + it + 1)

    return body(ab, src2, dst2, w2)


def _pad128(v):
    return jnp.concatenate([v, jnp.zeros_like(v)], axis=1)


def _tca(x, w1f, b1f, wc1):
    def body(x_r, w1_r, b1_r, wc1_r, u1_r):
        h = jnp.maximum(
            jnp.dot(x_r[...], w1_r[...], preferred_element_type=F32) + b1_r[...],
            0.0,
        )
        u1_r[...] = _pad128(jnp.dot(h, wc1_r[...], preferred_element_type=F32))

    grid = (N // RB,)
    return pl.pallas_call(
        body,
        grid=grid,
        in_specs=[
            pl.BlockSpec((RB, F_IN), lambda i: (i, 0)),
            pl.BlockSpec((F_IN, H), lambda i: (0, 0)),
            pl.BlockSpec((H,), lambda i: (0,)),
            pl.BlockSpec((H, H), lambda i: (0, 0)),
        ],
        out_specs=pl.BlockSpec((RB, W128), lambda i: (i, 0)),
        out_shape=jax.ShapeDtypeStruct((N, W128), F32),
    )(x, w1f, b1f, wc1)


def _tcb(u1, d0, d1):
    def body(u1_r, d0_r, d1_r, y1_r, dinv_r):
        deg = d0_r[...][:, 0:1] + d1_r[...][:, 0:1] + 1.0
        dinv = lax.rsqrt(deg)
        dinv_r[...] = dinv
        y1_r[...] = u1_r[...] * dinv

    grid = (N // RB,)
    return pl.pallas_call(
        body,
        grid=grid,
        in_specs=[
            pl.BlockSpec((RB, W128), lambda i: (i, 0)),
            pl.BlockSpec((RB, W128), lambda i: (i, 0)),
            pl.BlockSpec((RB, W128), lambda i: (i, 0)),
        ],
        out_specs=[
            pl.BlockSpec((RB, W128), lambda i: (i, 0)),
            pl.BlockSpec((RB, 1), lambda i: (i, 0)),
        ],
        out_shape=[
            jax.ShapeDtypeStruct((N, W128), F32),
            jax.ShapeDtypeStruct((N, 1), F32),
        ],
    )(u1, d0, d1)


def _tc2(p0, p1, y, dinv, bc, wnext):
    def body(p0_r, p1_r, y_r, dinv_r, bc_r, w_r, ynext_r):
        dinv = dinv_r[...]
        agg = (p0_r[...] + p1_r[...] + y_r[...])[:, :H]
        h = jnp.maximum(dinv * agg + bc_r[...], 0.0)
        ynext_r[...] = _pad128(
            jnp.dot(h, w_r[...], preferred_element_type=F32) * dinv
        )

    grid = (N // RB,)
    return pl.pallas_call(
        body,
        grid=grid,
        in_specs=[
            pl.BlockSpec((RB, W128), lambda i: (i, 0)),
            pl.BlockSpec((RB, W128), lambda i: (i, 0)),
            pl.BlockSpec((RB, W128), lambda i: (i, 0)),
            pl.BlockSpec((RB, 1), lambda i: (i, 0)),
            pl.BlockSpec((H,), lambda i: (0,)),
            pl.BlockSpec((H, H), lambda i: (0, 0)),
        ],
        out_specs=pl.BlockSpec((RB, W128), lambda i: (i, 0)),
        out_shape=jax.ShapeDtypeStruct((N, W128), F32),
    )(p0, p1, y, dinv, bc, wnext)


def _tc3(q0, q1, y2, dinv, bc2, wa, wb, cvec):
    def body(q0_r, q1_r, y_r, dinv_r, bc_r, wa_r, wb_r, c_r, h3_r, ab_r):
        dinv = dinv_r[...]
        agg = (q0_r[...] + q1_r[...] + y_r[...])[:, :H]
        h3 = jnp.maximum(dinv * agg + bc_r[...], 0.0)
        h3_r[...] = h3
        a2 = jnp.dot(h3, wa_r[...], preferred_element_type=F32) + c_r[...]
        b2 = jnp.dot(h3, wb_r[...], preferred_element_type=F32)
        ab_r[...] = jnp.concatenate([a2, b2], axis=1)

    grid = (N // RB,)
    return pl.pallas_call(
        body,
        grid=grid,
        in_specs=[
            pl.BlockSpec((RB, W128), lambda i: (i, 0)),
            pl.BlockSpec((RB, W128), lambda i: (i, 0)),
            pl.BlockSpec((RB, W128), lambda i: (i, 0)),
            pl.BlockSpec((RB, 1), lambda i: (i, 0)),
            pl.BlockSpec((H,), lambda i: (0,)),
            pl.BlockSpec((H, H), lambda i: (0, 0)),
            pl.BlockSpec((H, H), lambda i: (0, 0)),
            pl.BlockSpec((H,), lambda i: (0,)),
        ],
        out_specs=[
            pl.BlockSpec((RB, H), lambda i: (i, 0)),
            pl.BlockSpec((RB, W128), lambda i: (i, 0)),
        ],
        out_shape=[
            jax.ShapeDtypeStruct((N, H), F32),
            jax.ShapeDtypeStruct((N, W128), F32),
        ],
    )(q0, q1, y2, dinv, bc2, wa, wb, cvec)


def _tc4(s2, be2):
    EBR = 8192

    def body(s_r, b_r, out_r):
        t = jnp.sum(s_r[...], axis=1) + b_r[0]
        out_r[...] = (1.0 / (1.0 + jnp.exp(-t)))[:, None]

    grid = (E_PAD // EBR,)
    return pl.pallas_call(
        body,
        grid=grid,
        in_specs=[
            pl.BlockSpec((EBR, 16), lambda i: (i, 0)),
            pl.BlockSpec((1,), lambda i: (0,)),
        ],
        out_specs=pl.BlockSpec((EBR, 1), lambda i: (i, 0)),
        out_shape=jax.ShapeDtypeStruct((E_PAD, 1), F32),
    )(s2, be2)


def kernel(x, edge_index, W1, b1, bn1_g, bn1_b, Wc1, bc1, Wc2, bc2,
           We1, be1, bne_g, bne_b, We2, be2):
    npad = E_PAD - E
    spread = jnp.arange(npad, dtype=jnp.int32) % PAD_ROWS
    src2 = jnp.concatenate([edge_index[0], spread]).reshape(NEBP, EB)
    dst2 = jnp.concatenate([edge_index[1], N + spread]).reshape(NEBP, EB)

    s1 = bn1_g / jnp.sqrt(1.0 + EPS)
    w1f = W1 * s1[None, :]
    b1f = b1 * s1 + bn1_b
    sg = bne_g / jnp.sqrt(1.0 + EPS)
    wa = We1[:H] * sg[None, :]
    wb = We1[H:] * sg[None, :]
    cvec = be1 * sg + bne_b
    w2 = We2[:, 0]
    zeros128 = jnp.zeros((NT, W128), F32)
    ones_blk = jnp.ones((EB, W128), F32)

    u1 = _tca(x, w1f, b1f, Wc1)
    d0, d1 = _deg_sc(dst2, ones_blk, zeros128)
    y1, dinv = _tcb(u1, d0, d1)
    p0, p1 = _conv_sc(y1, src2, dst2, zeros128)
    y2 = _tc2(p0, p1, y1, dinv, bc1, Wc2)
    q0, q1 = _conv_sc(y2, src2, dst2, zeros128)
    h3, ab = _tc3(q0, q1, y2, dinv, bc2, wa, wb, cvec)
    s_flat = _head_sc(ab, src2, dst2, w2)
    pred = _tc4(s_flat.reshape(E_PAD, 16), be2)
    return (h3, pred[:E])
